# baseline (speedup 1.0000x reference)
import jax
import jax.numpy as jnp
from jax import lax
from jax.experimental import pallas as pl
from jax.experimental.pallas import tpu as pltpu

_U = 0.99
_R = 2.5
_TS = 16
_NH = 4
_M = 9
_CIN = 96
_COUT = 192
_HO = 112
_HW = 224
_P = 113
_QF = _P * _P
_QB = 1408
_NQ = 9
_QTOT = _NQ * _QB
_QPAD = _QTOT + 128
_NPATCH = _HO * _HO
_HPS = (0, 1, 0)
_DHS = (0, 0, 1)
_TAPS = tuple(
    (_HPS[kh] * 2 + _HPS[kw], _DHS[kh] * _P + _DHS[kw])
    for kh in range(3) for kw in range(3)
)


def _vote_body(dot_ref, kproj_ref, hb_ref, mask_ref):
    bn = dot_ref.shape[0]
    anyhit = jnp.zeros((1, _COUT), jnp.bool_)
    for h in range(_NH):
        z = dot_ref[:, h, :] + hb_ref[h, 0]
        zi = jnp.floor(z / _R).astype(jnp.int32)
        bk = jnp.abs(zi % _TS)
        bestc = jnp.sum(jnp.where(bk == 0, 1, 0))
        bestv = jnp.int32(0)
        for v in range(1, _TS):
            cv = jnp.sum(jnp.where(bk == v, 1, 0))
            upd = cv > bestc
            bestv = jnp.where(upd, jnp.int32(v), bestv)
            bestc = jnp.where(upd, cv, bestc)
        ki = jnp.abs(jnp.floor(kproj_ref[h:h + 1, :] / _R).astype(jnp.int32)
                     % _TS)
        anyhit = jnp.logical_or(anyhit, ki == bestv)
    mask_ref[...] = jnp.where(anyhit, jnp.float32(_NH / _TS), jnp.float32(0.))


def _conv_body(pf_ref, halo_ref, wt_ref, out_ref):
    xs = [jnp.concatenate([pf_ref[0, p], halo_ref[0, p]], axis=-1)
          for p in range(4)]
    acc = jnp.zeros((_COUT, _QB), jnp.float32)
    for t, (p, s) in enumerate(_TAPS):
        acc = acc + jnp.dot(wt_ref[t], xs[p][:, s:s + _QB],
                            preferred_element_type=jnp.float32)
    out_ref[0] = acc


def kernel(x, weight, hash_a, hash_b):
    bn = x.shape[0]
    f32 = jnp.float32

    w_flat = weight.reshape(_COUT, -1)
    denom = jnp.linalg.norm(w_flat, axis=1).max()
    w_u = _U * w_flat / denom
    norms = jnp.linalg.norm(w_u, axis=1, keepdims=True)
    powers = jnp.concatenate([norms ** (2 ** (i + 1)) for i in range(_M)],
                             axis=1)
    halves = jnp.full((_COUT, _M), 0.5, dtype=w_u.dtype)
    w_pq = jnp.concatenate([w_u, powers, halves], axis=1)
    k_proj = lax.stop_gradient(w_pq @ hash_a.T + hash_b[None, :])

    x_u = _U * x / denom
    q_chan = jnp.full((bn, 1, _HW, _HW), 0.5, dtype=x.dtype)
    p_chan = jnp.broadcast_to(
        (jnp.linalg.norm(x_u.reshape(bn, -1), axis=1) ** 2).reshape(bn, 1, 1, 1),
        (bn, 1, _HW, _HW)).astype(x.dtype)
    x_aug = jnp.concatenate([x_u, q_chan, p_chan], axis=1)
    hk = hash_a.reshape(_NH, _CIN + 2, 3, 3)
    dotted = lax.stop_gradient(lax.conv_general_dilated(
        x_aug, hk, window_strides=(2, 2), padding=((1, 1), (1, 1)),
        rhs_dilation=(1, 1), dimension_numbers=('NCHW', 'OIHW', 'NCHW')))

    mask = pl.pallas_call(
        _vote_body,
        out_shape=jax.ShapeDtypeStruct((1, _COUT), f32),
    )(dotted.reshape(bn, _NH, _NPATCH), k_proj.T, hash_b.reshape(_NH, 1))

    xp = jnp.pad(x, ((0, 0), (0, 0), (1, 1), (1, 1)))
    planes = xp.reshape(bn, _CIN, _P, 2, _P, 2).transpose(0, 3, 5, 1, 2, 4)
    pf = planes.reshape(bn, 4, _CIN, _QF)
    pf = jnp.pad(pf, ((0, 0), (0, 0), (0, 0), (0, _QPAD - _QF)))
    pf = pf.astype(jnp.bfloat16)
    wt = weight.transpose(2, 3, 0, 1).reshape(9, _COUT, _CIN)
    wt = (wt * mask.reshape(1, _COUT, 1)).astype(jnp.bfloat16)

    out = pl.pallas_call(
        _conv_body,
        grid=(bn, _NQ),
        in_specs=[
            pl.BlockSpec((1, 4, _CIN, _QB), lambda b, q: (b, 0, 0, q)),
            pl.BlockSpec((1, 4, _CIN, 128),
                         lambda b, q: (b, 0, 0, (q + 1) * (_QB // 128))),
            pl.BlockSpec((9, _COUT, _CIN), lambda b, q: (0, 0, 0)),
        ],
        out_specs=pl.BlockSpec((1, _COUT, _QB), lambda b, q: (b, 0, q)),
        out_shape=jax.ShapeDtypeStruct((bn, _COUT, _QTOT), f32),
        compiler_params=pltpu.CompilerParams(
            dimension_semantics=("parallel", "arbitrary")),
    )(pf, pf, wt)

    return out[:, :, :_HO * _P].reshape(bn, _COUT, _HO, _P)[:, :, :, :_HO]


# speedup vs baseline: 1.6185x; 1.6185x over previous
import jax
import jax.numpy as jnp
from jax import lax
from jax.experimental import pallas as pl
from jax.experimental.pallas import tpu as pltpu

_U = 0.99
_R = 2.5
_TS = 16
_NH = 4
_M = 9
_CIN = 96
_COUT = 192
_HO = 112
_HW = 224
_RB = 8
_NR = _HO // _RB
_NPATCH = _HO * _HO


def _vote_body(dot_ref, kproj_ref, hb_ref, mask_ref):
    anyhit = jnp.zeros((1, _COUT), jnp.bool_)
    for h in range(_NH):
        z = dot_ref[:, h, :] + hb_ref[h, 0]
        zi = jnp.floor(z / _R).astype(jnp.int32)
        bk = jnp.abs(zi % _TS)
        bestc = jnp.sum(jnp.where(bk == 0, 1, 0))
        bestv = jnp.int32(0)
        for v in range(1, _TS):
            cv = jnp.sum(jnp.where(bk == v, 1, 0))
            upd = cv > bestc
            bestv = jnp.where(upd, jnp.int32(v), bestv)
            bestc = jnp.where(upd, cv, bestc)
        ki = jnp.abs(jnp.floor(kproj_ref[h:h + 1, :] / _R).astype(jnp.int32)
                     % _TS)
        anyhit = jnp.logical_or(anyhit, ki == bestv)
    mask_ref[...] = jnp.where(anyhit, jnp.float32(_NH / _TS), jnp.float32(0.))


def _split_cols(v):
    n = v.shape[1]
    vr = v.reshape(_CIN, n, _HO, 2)
    vt = jnp.swapaxes(vr, 2, 3)
    return vt[:, :, 0, :], vt[:, :, 1, :]


def _conv_body(xa_ref, xh_ref, wt_ref, out_ref):
    r = pl.program_id(1)
    bf = jnp.bfloat16
    xa = xa_ref[0].astype(bf)
    xe0 = xa[:, :, :_HW]
    xe1 = xa[:, :, _HW:]
    h1 = jnp.where(r == 0, jnp.bfloat16(0.0),
                   xh_ref[0, :, _RB - 1:_RB, _HW:].astype(bf))
    b0 = jnp.concatenate([h1, xe1[:, :_RB - 1, :]], axis=1)
    rows = (b0, xe0, xe1)
    acc = jnp.zeros((_COUT, _RB, _HO), jnp.float32)
    for kh in range(3):
        e0, e1 = _split_cols(rows[kh])
        e1l = jnp.concatenate(
            [jnp.zeros((_CIN, _RB, 1), bf), e1[:, :, :_HO - 1]], axis=2)
        cols = (e1l, e0, e1)
        for kw in range(3):
            acc = acc + lax.dot_general(
                wt_ref[kh * 3 + kw], cols[kw],
                (((1,), (0,)), ((), ())),
                preferred_element_type=jnp.float32)
    out_ref[0] = acc


def kernel(x, weight, hash_a, hash_b):
    bn = x.shape[0]
    f32 = jnp.float32

    w_flat = weight.reshape(_COUT, -1)
    denom = jnp.linalg.norm(w_flat, axis=1).max()
    w_u = _U * w_flat / denom
    norms = jnp.linalg.norm(w_u, axis=1, keepdims=True)
    powers = jnp.concatenate([norms ** (2 ** (i + 1)) for i in range(_M)],
                             axis=1)
    halves = jnp.full((_COUT, _M), 0.5, dtype=w_u.dtype)
    w_pq = jnp.concatenate([w_u, powers, halves], axis=1)
    k_proj = lax.stop_gradient(w_pq @ hash_a.T + hash_b[None, :])

    x_u = _U * x / denom
    q_chan = jnp.full((bn, 1, _HW, _HW), 0.5, dtype=x.dtype)
    p_chan = jnp.broadcast_to(
        (jnp.linalg.norm(x_u.reshape(bn, -1), axis=1) ** 2).reshape(bn, 1, 1, 1),
        (bn, 1, _HW, _HW)).astype(x.dtype)
    x_aug = jnp.concatenate([x_u, q_chan, p_chan], axis=1)
    hk = hash_a.reshape(_NH, _CIN + 2, 3, 3)
    dotted = lax.stop_gradient(lax.conv_general_dilated(
        x_aug, hk, window_strides=(2, 2), padding=((1, 1), (1, 1)),
        rhs_dilation=(1, 1), dimension_numbers=('NCHW', 'OIHW', 'NCHW')))

    mask = pl.pallas_call(
        _vote_body,
        out_shape=jax.ShapeDtypeStruct((1, _COUT), f32),
    )(dotted.reshape(bn, _NH, _NPATCH), k_proj.T, hash_b.reshape(_NH, 1))

    wt = weight.transpose(2, 3, 0, 1).reshape(9, _COUT, _CIN)
    wt = (wt * mask.reshape(1, _COUT, 1)).astype(jnp.bfloat16)
    xv = x.reshape(bn, _CIN, _HO, 2 * _HW)

    out = pl.pallas_call(
        _conv_body,
        grid=(bn, _NR),
        in_specs=[
            pl.BlockSpec((1, _CIN, _RB, 2 * _HW), lambda b, r: (b, 0, r, 0)),
            pl.BlockSpec((1, _CIN, _RB, 2 * _HW),
                         lambda b, r: (b, 0, jnp.maximum(r - 1, 0), 0)),
            pl.BlockSpec((9, _COUT, _CIN), lambda b, r: (0, 0, 0)),
        ],
        out_specs=pl.BlockSpec((1, _COUT, _RB, _HO), lambda b, r: (b, 0, r, 0)),
        out_shape=jax.ShapeDtypeStruct((bn, _COUT, _HO, _HO), f32),
        compiler_params=pltpu.CompilerParams(
            dimension_semantics=("parallel", "arbitrary")),
    )(xv, xv, wt)

    return out


# speedup vs baseline: 3.0783x; 1.9020x over previous
import jax
import jax.numpy as jnp
from jax import lax
from jax.experimental import pallas as pl
from jax.experimental.pallas import tpu as pltpu

_U = 0.99
_R = 2.5
_TS = 16
_NH = 4
_M = 9
_CIN = 96
_COUT = 192
_HO = 112
_HW = 224
_RB = 8
_NR = _HO // _RB
_NPATCH = _HO * _HO


def _vote_body(dot_ref, kproj_ref, hb_ref, mask_ref):
    anyhit = jnp.zeros((1, _COUT), jnp.bool_)
    for h in range(_NH):
        z = dot_ref[:, h, :] + hb_ref[h, 0]
        zi = jnp.floor(z / _R).astype(jnp.int32)
        bk = jnp.abs(zi % _TS)
        bestc = jnp.sum(jnp.where(bk == 0, 1, 0))
        bestv = jnp.int32(0)
        for v in range(1, _TS):
            cv = jnp.sum(jnp.where(bk == v, 1, 0))
            upd = cv > bestc
            bestv = jnp.where(upd, jnp.int32(v), bestv)
            bestc = jnp.where(upd, cv, bestc)
        ki = jnp.abs(jnp.floor(kproj_ref[h:h + 1, :] / _R).astype(jnp.int32)
                     % _TS)
        anyhit = jnp.logical_or(anyhit, ki == bestv)
    mask_ref[...] = jnp.where(anyhit, jnp.float32(_NH / _TS), jnp.float32(0.))


def _aug_body(x_ref, denom_ref, p_ref, out_ref):
    bf = jnp.bfloat16
    d = denom_ref[0, 0]
    pb = p_ref[0, 0, 0]
    xv = (_U * x_ref[0]) / d
    ra = xv.shape[1]
    rows = []
    for r in range(ra):
        rows.append(jnp.swapaxes(xv[:, r, :], 0, 1).reshape(1, _HW, _CIN))
    xt = jnp.concatenate(rows, axis=0).astype(bf)
    qp = jnp.concatenate(
        [jnp.full((ra, _HW, 1), 0.5, bf),
         jnp.broadcast_to(pb.astype(bf), (ra, _HW, 1))], axis=2)
    out_ref[0] = jnp.concatenate([xt, qp], axis=2)


def _split_cols(v):
    n = v.shape[1]
    vr = v.reshape(_CIN, n, _HO, 2)
    vt = jnp.swapaxes(vr, 2, 3)
    return vt[:, :, 0, :], vt[:, :, 1, :]


def _conv_body(xa_ref, xh_ref, wt_ref, out_ref):
    r = pl.program_id(1)
    bf = jnp.bfloat16
    xa = xa_ref[0].astype(bf)
    xe0 = xa[:, :, :_HW]
    xe1 = xa[:, :, _HW:]
    h1 = jnp.where(r == 0, jnp.bfloat16(0.0),
                   xh_ref[0, :, _RB - 1:_RB, _HW:].astype(bf))
    b0 = jnp.concatenate([h1, xe1[:, :_RB - 1, :]], axis=1)
    rows = (b0, xe0, xe1)
    acc = jnp.zeros((_COUT, _RB, _HO), jnp.float32)
    for kh in range(3):
        e0, e1 = _split_cols(rows[kh])
        e1l = jnp.concatenate(
            [jnp.zeros((_CIN, _RB, 1), bf), e1[:, :, :_HO - 1]], axis=2)
        cols = (e1l, e0, e1)
        for kw in range(3):
            acc = acc + lax.dot_general(
                wt_ref[kh * 3 + kw], cols[kw],
                (((1,), (0,)), ((), ())),
                preferred_element_type=jnp.float32)
    out_ref[0] = acc


def kernel(x, weight, hash_a, hash_b):
    bn = x.shape[0]
    f32 = jnp.float32

    w_flat = weight.reshape(_COUT, -1)
    denom = jnp.linalg.norm(w_flat, axis=1).max()
    w_u = _U * w_flat / denom
    norms = jnp.linalg.norm(w_u, axis=1, keepdims=True)
    powers = jnp.concatenate([norms ** (2 ** (i + 1)) for i in range(_M)],
                             axis=1)
    halves = jnp.full((_COUT, _M), 0.5, dtype=w_u.dtype)
    w_pq = jnp.concatenate([w_u, powers, halves], axis=1)
    k_proj = lax.stop_gradient(w_pq @ hash_a.T + hash_b[None, :])

    x_u = _U * x / denom
    p = (jnp.linalg.norm(x_u.reshape(bn, -1), axis=1) ** 2).astype(x.dtype)
    hk = hash_a.reshape(_NH, _CIN + 2, 3, 3)

    _ra = 16
    x_aug = pl.pallas_call(
        _aug_body,
        grid=(bn, _HW // _ra),
        in_specs=[
            pl.BlockSpec((1, _CIN, _ra, _HW), lambda b, r: (b, 0, r, 0)),
            pl.BlockSpec((1, 1), lambda b, r: (0, 0)),
            pl.BlockSpec((1, 1, 1), lambda b, r: (b, 0, 0)),
        ],
        out_specs=pl.BlockSpec((1, _ra, _HW, _CIN + 2),
                               lambda b, r: (b, r, 0, 0)),
        out_shape=jax.ShapeDtypeStruct((bn, _HW, _HW, _CIN + 2), jnp.bfloat16),
        compiler_params=pltpu.CompilerParams(
            dimension_semantics=("parallel", "parallel")),
    )(x, denom.reshape(1, 1), p.reshape(bn, 1, 1))

    dotted = lax.stop_gradient(lax.conv_general_dilated(
        x_aug, hk.astype(jnp.bfloat16), window_strides=(2, 2),
        padding=((1, 1), (1, 1)), rhs_dilation=(1, 1),
        dimension_numbers=('NHWC', 'OIHW', 'NCHW'),
        preferred_element_type=jnp.float32))

    mask = pl.pallas_call(
        _vote_body,
        out_shape=jax.ShapeDtypeStruct((1, _COUT), f32),
    )(dotted.reshape(bn, _NH, _NPATCH), k_proj.T, hash_b.reshape(_NH, 1))

    wt = weight.transpose(2, 3, 0, 1).reshape(9, _COUT, _CIN)
    wt = (wt * mask.reshape(1, _COUT, 1)).astype(jnp.bfloat16)
    xv = x.reshape(bn, _CIN, _HO, 2 * _HW)

    out = pl.pallas_call(
        _conv_body,
        grid=(bn, _NR),
        in_specs=[
            pl.BlockSpec((1, _CIN, _RB, 2 * _HW), lambda b, r: (b, 0, r, 0)),
            pl.BlockSpec((1, _CIN, _RB, 2 * _HW),
                         lambda b, r: (b, 0, jnp.maximum(r - 1, 0), 0)),
            pl.BlockSpec((9, _COUT, _CIN), lambda b, r: (0, 0, 0)),
        ],
        out_specs=pl.BlockSpec((1, _COUT, _RB, _HO), lambda b, r: (b, 0, r, 0)),
        out_shape=jax.ShapeDtypeStruct((bn, _COUT, _HO, _HO), f32),
        compiler_params=pltpu.CompilerParams(
            dimension_semantics=("parallel", "arbitrary")),
    )(xv, xv, wt)

    return out
